# Initial kernel scaffold; baseline (speedup 1.0000x reference)
import jax
import jax.numpy as jnp
from jax.experimental import pallas as pl


def kernel(x_1, edge_index, W1, W2):
    raise NotImplementedError("write your pallas kernel here")


# speedup vs baseline: 3.5176x; 3.5176x over previous
import functools

import jax
import jax.numpy as jnp
from jax import lax
from jax.experimental import pallas as pl
from jax.experimental.pallas import tpu as pltpu
from jax.experimental.pallas import tpu_sc as plsc

N_NODES = 10000
N_EDGES = 320000
CH = 128
NC = 2
NS = 16
NW = NC * NS
EW = N_EDGES // NW
CHUNK = 80
NCHUNK = EW // CHUNK
N_PAD = 10240
NR = N_PAD // NS
DR = N_PAD // CH

_mesh = plsc.VectorSubcoreMesh(core_axis_name="c", subcore_axis_name="s")


def _zero_vmem_2d(ref, rows, cols):
    zv = jnp.zeros((16,), jnp.float32)
    cpr = cols // 16

    def body(k, _):
        r = k // cpr
        c = k % cpr
        ref[r, pl.ds(c * 16, 16)] = zv
        return 0

    lax.fori_loop(0, rows * cpr, body, 0)


@functools.partial(
    pl.kernel,
    out_type=(
        jax.ShapeDtypeStruct((NC * N_PAD, CH), jnp.float32),
        jax.ShapeDtypeStruct((NW * DR, CH), jnp.float32),
    ),
    mesh=_mesh,
    scratch_types=[
        pltpu.VMEM_SHARED((N_PAD, CH), jnp.float32),
        pltpu.VMEM((CHUNK, CH), jnp.float32),
        pltpu.VMEM((DR, CH), jnp.float32),
        pltpu.VMEM((CHUNK,), jnp.int32),
        pltpu.VMEM((CHUNK,), jnp.int32),
    ],
    compiler_params=pltpu.CompilerParams(needs_layout_passes=False),
)
def _sc_scatter(x_hbm, src_hbm, dst_hbm, pagg_hbm, pdeg_hbm,
                agg_s, xbuf, degtab, sidx, didx):
    c = lax.axis_index("c")
    s = lax.axis_index("s")
    wid = c * NS + s
    nbase = s * NR

    _zero_vmem_2d(degtab, DR, CH)
    _zero_vmem_2d(xbuf, CHUNK, CH)
    for j in range(NR // CHUNK):
        pltpu.sync_copy(xbuf, agg_s.at[pl.ds(nbase + j * CHUNK, CHUNK)])
    plsc.subcore_barrier()

    onev = jnp.ones((16,), jnp.float32)

    def count16(idxref, k):
        idx = idxref[pl.ds(k * 16, 16)]
        r16 = lax.shift_right_logical(idx, 7)
        c16 = jnp.bitwise_and(idx, 127)
        plsc.addupdate_scatter(degtab, [r16, c16], onev)

    def chunk_body(i, _):
        base = wid * EW + i * CHUNK
        pltpu.sync_copy(src_hbm.at[pl.ds(base, CHUNK)], sidx)
        pltpu.sync_copy(dst_hbm.at[pl.ds(base, CHUNK)], didx)
        pltpu.sync_copy(x_hbm.at[pl.ds(base, CHUNK)], xbuf)
        pltpu.sync_copy(xbuf, agg_s.at[sidx], add=True)
        pltpu.sync_copy(xbuf, agg_s.at[didx], add=True)
        for k in range(CHUNK // 16):
            count16(sidx, k)
            count16(didx, k)
        return 0

    lax.fori_loop(0, NCHUNK, chunk_body, 0)
    plsc.subcore_barrier()

    pltpu.sync_copy(agg_s.at[pl.ds(nbase, NR)],
                    pagg_hbm.at[pl.ds(c * N_PAD + nbase, NR)])
    pltpu.sync_copy(degtab, pdeg_hbm.at[pl.ds(wid * DR, DR)])


def _tc_middle_body(pagg_ref, pdeg_ref, w1_ref, w2_ref, g_ref):
    agg = pagg_ref[0:N_NODES, :] + pagg_ref[N_PAD:N_PAD + N_NODES, :]
    degp = pdeg_ref[0:DR, :]
    for w in range(1, NW):
        degp = degp + pdeg_ref[w * DR:(w + 1) * DR, :]
    deg = jnp.reshape(degp, (N_PAD,))[0:N_NODES][:, None]
    deg = jnp.where(deg > 0.0, deg, 1.0)
    z = jax.lax.dot(agg, w1_ref[...], precision=jax.lax.Precision.HIGHEST,
                    preferred_element_type=jnp.float32) / deg
    x0 = 1.0 / (1.0 + jnp.exp(-z))
    m2 = jax.lax.dot(x0, w2_ref[...], precision=jax.lax.Precision.HIGHEST,
                     preferred_element_type=jnp.float32)
    g_ref[...] = jnp.exp(m2 * -0.5)


def _tc_middle(pagg, pdeg, W1, W2):
    return pl.pallas_call(
        _tc_middle_body,
        out_shape=jax.ShapeDtypeStruct((N_NODES, CH), jnp.float32),
    )(pagg, pdeg, W1, W2)


@functools.partial(
    pl.kernel,
    out_type=jax.ShapeDtypeStruct((N_EDGES, CH), jnp.float32),
    mesh=_mesh,
    scratch_types=[
        pltpu.VMEM((CHUNK,), jnp.int32),
        pltpu.VMEM((CHUNK,), jnp.int32),
        pltpu.VMEM((CHUNK, CH), jnp.float32),
        pltpu.VMEM((CHUNK, CH), jnp.float32),
        pltpu.VMEM((CHUNK, CH), jnp.float32),
    ],
    compiler_params=pltpu.CompilerParams(needs_layout_passes=False),
)
def _sc_gather(g_hbm, src_hbm, dst_hbm, out_hbm, sidx, didx, abuf, bbuf, obuf):
    c = lax.axis_index("c")
    s = lax.axis_index("s")
    wid = c * NS + s

    def chunk_body(i, _):
        base = wid * EW + i * CHUNK
        pltpu.sync_copy(src_hbm.at[pl.ds(base, CHUNK)], sidx)
        pltpu.sync_copy(dst_hbm.at[pl.ds(base, CHUNK)], didx)
        pltpu.sync_copy(g_hbm.at[sidx], abuf)
        pltpu.sync_copy(g_hbm.at[didx], bbuf)

        def row_body(r, _):
            for cc in range(CH // 16):
                a = abuf[r, pl.ds(cc * 16, 16)]
                b = bbuf[r, pl.ds(cc * 16, 16)]
                obuf[r, pl.ds(cc * 16, 16)] = 1.0 / (1.0 + a * b)
            return 0

        lax.fori_loop(0, CHUNK, row_body, 0)
        pltpu.sync_copy(obuf, out_hbm.at[pl.ds(base, CHUNK)])
        return 0

    lax.fori_loop(0, NCHUNK, chunk_body, 0)


def kernel(x_1, edge_index, W1, W2):
    src = edge_index[0]
    dst = edge_index[1]
    pagg, pdeg = _sc_scatter(x_1, src, dst)
    g = _tc_middle(pagg, pdeg, W1, W2)
    return _sc_gather(g, src, dst)


# speedup vs baseline: 4.2987x; 1.2221x over previous
import functools

import jax
import jax.numpy as jnp
from jax import lax
from jax.experimental import pallas as pl
from jax.experimental.pallas import tpu as pltpu
from jax.experimental.pallas import tpu_sc as plsc

N_NODES = 10000
N_EDGES = 320000
CH = 128
NC = 2
NS = 16
NW = NC * NS
EW = N_EDGES // NW
CHUNK = 80
NCHUNK = EW // CHUNK
N_PAD = 10240
NR = N_PAD // NS
DR = N_PAD // CH

_mesh = plsc.VectorSubcoreMesh(core_axis_name="c", subcore_axis_name="s")


def _zero_vmem_2d(ref, rows, cols):
    zv = jnp.zeros((16,), jnp.float32)
    cpr = cols // 16

    def body(k, _):
        r = k // cpr
        c = k % cpr
        ref[r, pl.ds(c * 16, 16)] = zv
        return 0

    lax.fori_loop(0, rows * cpr, body, 0)


@functools.partial(
    pl.kernel,
    out_type=(
        jax.ShapeDtypeStruct((NC * N_PAD, CH), jnp.float32),
        jax.ShapeDtypeStruct((NW * DR, CH), jnp.float32),
    ),
    mesh=_mesh,
    scratch_types=[
        pltpu.VMEM_SHARED((N_PAD, CH), jnp.float32),
        pltpu.VMEM((CHUNK, CH), jnp.float32),
        pltpu.VMEM((DR, CH), jnp.float32),
        pltpu.VMEM((CHUNK,), jnp.int32),
        pltpu.VMEM((CHUNK,), jnp.int32),
    ],
    compiler_params=pltpu.CompilerParams(needs_layout_passes=False),
)
def _sc_scatter(x_hbm, src_hbm, dst_hbm, pagg_hbm, pdeg_hbm,
                agg_s, xbuf, degtab, sidx, didx):
    c = lax.axis_index("c")
    s = lax.axis_index("s")
    wid = c * NS + s
    nbase = s * NR

    _zero_vmem_2d(degtab, DR, CH)
    _zero_vmem_2d(xbuf, CHUNK, CH)
    for j in range(NR // CHUNK):
        pltpu.sync_copy(xbuf, agg_s.at[pl.ds(nbase + j * CHUNK, CHUNK)])
    plsc.subcore_barrier()

    onev = jnp.ones((16,), jnp.float32)

    def count16(idxref, k):
        idx = idxref[pl.ds(k * 16, 16)]
        r16 = lax.shift_right_logical(idx, 7)
        c16 = jnp.bitwise_and(idx, 127)
        plsc.addupdate_scatter(degtab, [r16, c16], onev)

    def chunk_body(i, _):
        base = wid * EW + i * CHUNK
        pltpu.sync_copy(src_hbm.at[pl.ds(base, CHUNK)], sidx)
        pltpu.sync_copy(dst_hbm.at[pl.ds(base, CHUNK)], didx)
        pltpu.sync_copy(x_hbm.at[pl.ds(base, CHUNK)], xbuf)
        pltpu.sync_copy(xbuf, agg_s.at[sidx], add=True)
        pltpu.sync_copy(xbuf, agg_s.at[didx], add=True)
        for k in range(CHUNK // 16):
            count16(sidx, k)
            count16(didx, k)
        return 0

    lax.fori_loop(0, NCHUNK, chunk_body, 0)
    plsc.subcore_barrier()

    pltpu.sync_copy(agg_s.at[pl.ds(nbase, NR)],
                    pagg_hbm.at[pl.ds(c * N_PAD + nbase, NR)])
    pltpu.sync_copy(degtab, pdeg_hbm.at[pl.ds(wid * DR, DR)])


def _tc_middle_body(pagg_ref, pdeg_ref, w1_ref, w2_ref, g_ref):
    agg = pagg_ref[0:N_NODES, :] + pagg_ref[N_PAD:N_PAD + N_NODES, :]
    degp = pdeg_ref[0:DR, :]
    for w in range(1, NW):
        degp = degp + pdeg_ref[w * DR:(w + 1) * DR, :]
    deg = jnp.reshape(degp, (N_PAD,))[0:N_NODES][:, None]
    deg = jnp.where(deg > 0.0, deg, 1.0)
    z = jax.lax.dot(agg, w1_ref[...], precision=jax.lax.Precision.HIGHEST,
                    preferred_element_type=jnp.float32) / deg
    x0 = 1.0 / (1.0 + jnp.exp(-z))
    m2 = jax.lax.dot(x0, w2_ref[...], precision=jax.lax.Precision.HIGHEST,
                     preferred_element_type=jnp.float32)
    g_ref[0:N_NODES, :] = jnp.exp(m2 * -0.5)
    g_ref[N_NODES:N_PAD, :] = jnp.zeros((N_PAD - N_NODES, CH), jnp.float32)


def _tc_middle(pagg, pdeg, W1, W2):
    return pl.pallas_call(
        _tc_middle_body,
        out_shape=jax.ShapeDtypeStruct((N_PAD, CH), jnp.float32),
    )(pagg, pdeg, W1, W2)


@functools.partial(
    pl.kernel,
    out_type=jax.ShapeDtypeStruct((N_EDGES, CH), jnp.float32),
    mesh=_mesh,
    scratch_types=[
        pltpu.VMEM_SHARED((N_PAD, CH), jnp.float32),
        pltpu.VMEM((CHUNK,), jnp.int32),
        pltpu.VMEM((CHUNK,), jnp.int32),
        pltpu.VMEM((CHUNK, CH), jnp.float32),
        pltpu.VMEM((CHUNK, CH), jnp.float32),
        pltpu.VMEM((CHUNK, CH), jnp.float32),
        pltpu.SemaphoreType.DMA,
        pltpu.SemaphoreType.DMA,
    ],
    compiler_params=pltpu.CompilerParams(needs_layout_passes=False),
)
def _sc_gather(g_hbm, src_hbm, dst_hbm, out_hbm,
               g_s, sidx, didx, abuf, bbuf, obuf, sema, semb):
    c = lax.axis_index("c")
    s = lax.axis_index("s")
    wid = c * NS + s
    nbase = s * NR

    pltpu.sync_copy(g_hbm.at[pl.ds(nbase, NR)], g_s.at[pl.ds(nbase, NR)])
    plsc.subcore_barrier()

    def chunk_body(i, _):
        base = wid * EW + i * CHUNK
        pltpu.sync_copy(src_hbm.at[pl.ds(base, CHUNK)], sidx)
        pltpu.sync_copy(dst_hbm.at[pl.ds(base, CHUNK)], didx)
        ca = pltpu.async_copy(g_s.at[sidx], abuf, sema)
        cb = pltpu.async_copy(g_s.at[didx], bbuf, semb)
        ca.wait()
        cb.wait()

        def row_body(r, _):
            for cc in range(CH // 16):
                a = abuf[r, pl.ds(cc * 16, 16)]
                b = bbuf[r, pl.ds(cc * 16, 16)]
                obuf[r, pl.ds(cc * 16, 16)] = 1.0 / (1.0 + a * b)
            return 0

        lax.fori_loop(0, CHUNK, row_body, 0)
        pltpu.sync_copy(obuf, out_hbm.at[pl.ds(base, CHUNK)])
        return 0

    lax.fori_loop(0, NCHUNK, chunk_body, 0)


def kernel(x_1, edge_index, W1, W2):
    src = edge_index[0]
    dst = edge_index[1]
    pagg, pdeg = _sc_scatter(x_1, src, dst)
    g = _tc_middle(pagg, pdeg, W1, W2)
    return _sc_gather(g, src, dst)


# speedup vs baseline: 5.4141x; 1.2595x over previous
import functools

import jax
import jax.numpy as jnp
from jax import lax
from jax.experimental import pallas as pl
from jax.experimental.pallas import tpu as pltpu
from jax.experimental.pallas import tpu_sc as plsc

N_NODES = 10000
N_EDGES = 320000
CH = 128
NC = 2
NS = 16
NW = NC * NS
EW = N_EDGES // NW
CHUNK = 240
NFULL = 41
TAIL = EW - NFULL * CHUNK
CHUNK_G = 184
NFULL_G = 54
TAIL_G = EW - NFULL_G * CHUNK_G
N_PAD = 10240
NR = N_PAD // NS
DR = N_PAD // CH

_mesh = plsc.VectorSubcoreMesh(core_axis_name="c", subcore_axis_name="s")


def _zero_vmem_2d(ref, rows, cols):
    zv = jnp.zeros((16,), jnp.float32)
    cpr = cols // 16

    def body(k, _):
        r = k // cpr
        c = k % cpr
        ref[r, pl.ds(c * 16, 16)] = zv
        return 0

    lax.fori_loop(0, rows * cpr, body, 0)


@functools.partial(
    pl.kernel,
    out_type=(
        jax.ShapeDtypeStruct((NC * N_PAD, CH), jnp.float32),
        jax.ShapeDtypeStruct((NW * DR, CH), jnp.float32),
    ),
    mesh=_mesh,
    scratch_types=[
        pltpu.VMEM_SHARED((N_PAD, CH), jnp.float32),
        pltpu.VMEM((CHUNK, CH), jnp.float32),
        pltpu.VMEM((DR, CH), jnp.float32),
        pltpu.VMEM((CHUNK,), jnp.int32),
        pltpu.VMEM((CHUNK,), jnp.int32),
    ],
    compiler_params=pltpu.CompilerParams(needs_layout_passes=False),
)
def _sc_scatter(x_hbm, src_hbm, dst_hbm, pagg_hbm, pdeg_hbm,
                agg_s, xbuf, degtab, sidx, didx):
    c = lax.axis_index("c")
    s = lax.axis_index("s")
    wid = c * NS + s
    nbase = s * NR

    _zero_vmem_2d(degtab, DR, CH)
    _zero_vmem_2d(xbuf, CHUNK, CH)
    pltpu.sync_copy(xbuf, agg_s.at[pl.ds(nbase, CHUNK)])
    pltpu.sync_copy(xbuf, agg_s.at[pl.ds(nbase + CHUNK, CHUNK)])
    pltpu.sync_copy(xbuf.at[pl.ds(0, NR - 2 * CHUNK)],
                    agg_s.at[pl.ds(nbase + 2 * CHUNK, NR - 2 * CHUNK)])
    plsc.subcore_barrier()

    onev = jnp.ones((16,), jnp.float32)

    def count16(idxref, k):
        idx = idxref[pl.ds(k * 16, 16)]
        r16 = lax.shift_right_logical(idx, 7)
        c16 = jnp.bitwise_and(idx, 127)
        plsc.addupdate_scatter(degtab, [r16, c16], onev)

    def do_chunk(base, n):
        sx = sidx.at[pl.ds(0, n)]
        dx = didx.at[pl.ds(0, n)]
        xb = xbuf.at[pl.ds(0, n)]
        pltpu.sync_copy(src_hbm.at[pl.ds(base, n)], sx)
        pltpu.sync_copy(dst_hbm.at[pl.ds(base, n)], dx)
        pltpu.sync_copy(x_hbm.at[pl.ds(base, n)], xb)
        pltpu.sync_copy(xb, agg_s.at[sx], add=True)
        pltpu.sync_copy(xb, agg_s.at[dx], add=True)

        def count_body(k, _):
            count16(sidx, k)
            count16(didx, k)
            return 0

        lax.fori_loop(0, n // 16, count_body, 0)

    def chunk_body(i, _):
        do_chunk(wid * EW + i * CHUNK, CHUNK)
        return 0

    lax.fori_loop(0, NFULL, chunk_body, 0)
    do_chunk(wid * EW + NFULL * CHUNK, TAIL)
    plsc.subcore_barrier()

    pltpu.sync_copy(agg_s.at[pl.ds(nbase, NR)],
                    pagg_hbm.at[pl.ds(c * N_PAD + nbase, NR)])
    pltpu.sync_copy(degtab, pdeg_hbm.at[pl.ds(wid * DR, DR)])


def _tc_middle_body(pagg_ref, pdeg_ref, w1_ref, w2_ref, g_ref):
    agg = pagg_ref[0:N_NODES, :] + pagg_ref[N_PAD:N_PAD + N_NODES, :]
    degp = pdeg_ref[0:DR, :]
    for w in range(1, NW):
        degp = degp + pdeg_ref[w * DR:(w + 1) * DR, :]
    deg = jnp.reshape(degp, (N_PAD,))[0:N_NODES][:, None]
    deg = jnp.where(deg > 0.0, deg, 1.0)
    z = jax.lax.dot(agg, w1_ref[...], precision=jax.lax.Precision.HIGHEST,
                    preferred_element_type=jnp.float32) / deg
    x0 = 1.0 / (1.0 + jnp.exp(-z))
    m2 = jax.lax.dot(x0, w2_ref[...], precision=jax.lax.Precision.HIGHEST,
                     preferred_element_type=jnp.float32)
    g_ref[0:N_NODES, :] = jnp.exp(m2 * -0.5)
    g_ref[N_NODES:N_PAD, :] = jnp.zeros((N_PAD - N_NODES, CH), jnp.float32)


def _tc_middle(pagg, pdeg, W1, W2):
    return pl.pallas_call(
        _tc_middle_body,
        out_shape=jax.ShapeDtypeStruct((N_PAD, CH), jnp.float32),
    )(pagg, pdeg, W1, W2)


@functools.partial(
    pl.kernel,
    out_type=jax.ShapeDtypeStruct((N_EDGES, CH), jnp.float32),
    mesh=_mesh,
    scratch_types=[
        pltpu.VMEM_SHARED((N_PAD, CH), jnp.float32),
        pltpu.VMEM((CHUNK_G,), jnp.int32),
        pltpu.VMEM((CHUNK_G,), jnp.int32),
        pltpu.VMEM((CHUNK_G, CH), jnp.float32),
        pltpu.VMEM((CHUNK_G, CH), jnp.float32),
        pltpu.SemaphoreType.DMA,
        pltpu.SemaphoreType.DMA,
    ],
    compiler_params=pltpu.CompilerParams(needs_layout_passes=False),
)
def _sc_gather(g_hbm, src_hbm, dst_hbm, out_hbm,
               g_s, sidx, didx, abuf, bbuf, sema, semb):
    c = lax.axis_index("c")
    s = lax.axis_index("s")
    wid = c * NS + s
    nbase = s * NR

    pltpu.sync_copy(g_hbm.at[pl.ds(nbase, NR)], g_s.at[pl.ds(nbase, NR)])
    plsc.subcore_barrier()

    def do_chunk(base, n):
        sx = sidx.at[pl.ds(0, n)]
        dx = didx.at[pl.ds(0, n)]
        ab = abuf.at[pl.ds(0, n)]
        bb = bbuf.at[pl.ds(0, n)]
        pltpu.sync_copy(src_hbm.at[pl.ds(base, n)], sx)
        pltpu.sync_copy(dst_hbm.at[pl.ds(base, n)], dx)
        ca = pltpu.async_copy(g_s.at[sx], ab, sema)
        cb = pltpu.async_copy(g_s.at[dx], bb, semb)
        ca.wait()
        cb.wait()

        def row_body(r, _):
            for cc in range(CH // 16):
                a = abuf[r, pl.ds(cc * 16, 16)]
                b = bbuf[r, pl.ds(cc * 16, 16)]
                abuf[r, pl.ds(cc * 16, 16)] = 1.0 / (1.0 + a * b)
            return 0

        lax.fori_loop(0, n, row_body, 0)
        pltpu.sync_copy(ab, out_hbm.at[pl.ds(base, n)])

    def chunk_body(i, _):
        do_chunk(wid * EW + i * CHUNK_G, CHUNK_G)
        return 0

    lax.fori_loop(0, NFULL_G, chunk_body, 0)
    do_chunk(wid * EW + NFULL_G * CHUNK_G, TAIL_G)


def kernel(x_1, edge_index, W1, W2):
    src = edge_index[0]
    dst = edge_index[1]
    pagg, pdeg = _sc_scatter(x_1, src, dst)
    g = _tc_middle(pagg, pdeg, W1, W2)
    return _sc_gather(g, src, dst)


# speedup vs baseline: 7.0732x; 1.3064x over previous
import functools

import jax
import jax.numpy as jnp
from jax import lax
from jax.experimental import pallas as pl
from jax.experimental.pallas import tpu as pltpu
from jax.experimental.pallas import tpu_sc as plsc

N_NODES = 10000
N_EDGES = 320000
CH = 128
NC = 2
NS = 16
NW = NC * NS
EW = N_EDGES // NW
CHUNK = 240
NFULL = 41
TAIL = EW - NFULL * CHUNK
CHUNK_G = 80
NCHUNK_G = EW // CHUNK_G
N_PAD = 10240
NR = N_PAD // NS
DR = N_PAD // CH

_mesh = plsc.VectorSubcoreMesh(core_axis_name="c", subcore_axis_name="s")


def _zero_vmem_2d(ref, rows, cols):
    zv = jnp.zeros((16,), jnp.float32)
    cpr = cols // 16

    def body(k, _):
        r = k // cpr
        c = k % cpr
        ref[r, pl.ds(c * 16, 16)] = zv
        return 0

    lax.fori_loop(0, rows * cpr, body, 0)


@functools.partial(
    pl.kernel,
    out_type=(
        jax.ShapeDtypeStruct((NC * N_PAD, CH), jnp.float32),
        jax.ShapeDtypeStruct((NW * DR, CH), jnp.float32),
    ),
    mesh=_mesh,
    scratch_types=[
        pltpu.VMEM_SHARED((N_PAD, CH), jnp.float32),
        pltpu.VMEM((CHUNK, CH), jnp.float32),
        pltpu.VMEM((DR, CH), jnp.float32),
        pltpu.VMEM((CHUNK,), jnp.int32),
        pltpu.VMEM((CHUNK,), jnp.int32),
    ],
    compiler_params=pltpu.CompilerParams(needs_layout_passes=False),
)
def _sc_scatter(x_hbm, src_hbm, dst_hbm, pagg_hbm, pdeg_hbm,
                agg_s, xbuf, degtab, sidx, didx):
    c = lax.axis_index("c")
    s = lax.axis_index("s")
    wid = c * NS + s
    nbase = s * NR

    _zero_vmem_2d(degtab, DR, CH)
    _zero_vmem_2d(xbuf, CHUNK, CH)
    pltpu.sync_copy(xbuf, agg_s.at[pl.ds(nbase, CHUNK)])
    pltpu.sync_copy(xbuf, agg_s.at[pl.ds(nbase + CHUNK, CHUNK)])
    pltpu.sync_copy(xbuf.at[pl.ds(0, NR - 2 * CHUNK)],
                    agg_s.at[pl.ds(nbase + 2 * CHUNK, NR - 2 * CHUNK)])
    plsc.subcore_barrier()

    onev = jnp.ones((16,), jnp.float32)

    def count16(idxref, k):
        idx = idxref[pl.ds(k * 16, 16)]
        r16 = lax.shift_right_logical(idx, 7)
        c16 = jnp.bitwise_and(idx, 127)
        plsc.addupdate_scatter(degtab, [r16, c16], onev)

    def do_chunk(base, n):
        sx = sidx.at[pl.ds(0, n)]
        dx = didx.at[pl.ds(0, n)]
        xb = xbuf.at[pl.ds(0, n)]
        pltpu.sync_copy(src_hbm.at[pl.ds(base, n)], sx)
        pltpu.sync_copy(dst_hbm.at[pl.ds(base, n)], dx)
        pltpu.sync_copy(x_hbm.at[pl.ds(base, n)], xb)
        pltpu.sync_copy(xb, agg_s.at[sx], add=True)
        pltpu.sync_copy(xb, agg_s.at[dx], add=True)

        def count_body(k, _):
            count16(sidx, k)
            count16(didx, k)
            return 0

        lax.fori_loop(0, n // 16, count_body, 0)

    def chunk_body(i, _):
        do_chunk(wid * EW + i * CHUNK, CHUNK)
        return 0

    lax.fori_loop(0, NFULL, chunk_body, 0)
    do_chunk(wid * EW + NFULL * CHUNK, TAIL)
    plsc.subcore_barrier()

    pltpu.sync_copy(agg_s.at[pl.ds(nbase, NR)],
                    pagg_hbm.at[pl.ds(c * N_PAD + nbase, NR)])
    pltpu.sync_copy(degtab, pdeg_hbm.at[pl.ds(wid * DR, DR)])


def _tc_middle_body(pagg_ref, pdeg_ref, w1_ref, w2_ref, g_ref):
    agg = pagg_ref[0:N_NODES, :] + pagg_ref[N_PAD:N_PAD + N_NODES, :]
    degp = pdeg_ref[0:DR, :]
    for w in range(1, NW):
        degp = degp + pdeg_ref[w * DR:(w + 1) * DR, :]
    deg = jnp.reshape(degp, (N_PAD,))[0:N_NODES][:, None]
    deg = jnp.where(deg > 0.0, deg, 1.0)
    z = jax.lax.dot(agg, w1_ref[...], precision=jax.lax.Precision.HIGHEST,
                    preferred_element_type=jnp.float32) / deg
    x0 = 1.0 / (1.0 + jnp.exp(-z))
    m2 = jax.lax.dot(x0, w2_ref[...], precision=jax.lax.Precision.HIGHEST,
                     preferred_element_type=jnp.float32)
    g_ref[0:N_NODES, :] = jnp.exp(m2 * -0.5)
    g_ref[N_NODES:N_PAD, :] = jnp.zeros((N_PAD - N_NODES, CH), jnp.float32)


def _tc_middle(pagg, pdeg, W1, W2):
    return pl.pallas_call(
        _tc_middle_body,
        out_shape=jax.ShapeDtypeStruct((N_PAD, CH), jnp.float32),
    )(pagg, pdeg, W1, W2)


@functools.partial(
    pl.kernel,
    out_type=jax.ShapeDtypeStruct((N_EDGES, CH), jnp.float32),
    mesh=_mesh,
    scratch_types=[
        pltpu.VMEM_SHARED((N_PAD, CH), jnp.float32),
        pltpu.VMEM((2, CHUNK_G), jnp.int32),
        pltpu.VMEM((2, CHUNK_G), jnp.int32),
        pltpu.VMEM((2, CHUNK_G, CH), jnp.float32),
        pltpu.VMEM((2, CHUNK_G, CH), jnp.float32),
        pltpu.SemaphoreType.DMA((2,)),
        pltpu.SemaphoreType.DMA((2,)),
        pltpu.SemaphoreType.DMA((2,)),
    ],
    compiler_params=pltpu.CompilerParams(needs_layout_passes=False),
)
def _sc_gather(g_hbm, src_hbm, dst_hbm, out_hbm,
               g_s, sidx, didx, abuf, bbuf, sem_g, sem_i, sem_w):
    c = lax.axis_index("c")
    s = lax.axis_index("s")
    wid = c * NS + s
    nbase = s * NR
    ebase = wid * EW

    pltpu.sync_copy(g_hbm.at[pl.ds(nbase, NR)], g_s.at[pl.ds(nbase, NR)])
    plsc.subcore_barrier()

    def start_idx(i, p):
        base = ebase + i * CHUNK_G
        pltpu.async_copy(src_hbm.at[pl.ds(base, CHUNK_G)], sidx.at[p],
                         sem_i.at[p])
        pltpu.async_copy(dst_hbm.at[pl.ds(base, CHUNK_G)], didx.at[p],
                         sem_i.at[p])

    def drain_idx(p):
        pltpu.make_async_copy(src_hbm.at[pl.ds(0, CHUNK_G)], sidx.at[p],
                              sem_i.at[p]).wait()
        pltpu.make_async_copy(dst_hbm.at[pl.ds(0, CHUNK_G)], didx.at[p],
                              sem_i.at[p]).wait()

    def start_gathers(p):
        pltpu.async_copy(g_s.at[sidx.at[p]], abuf.at[p], sem_g.at[p])
        pltpu.async_copy(g_s.at[didx.at[p]], bbuf.at[p], sem_g.at[p])

    def drain_gathers(p):
        pltpu.make_async_copy(g_s.at[sidx.at[p]], abuf.at[p],
                              sem_g.at[p]).wait()
        pltpu.make_async_copy(g_s.at[didx.at[p]], bbuf.at[p],
                              sem_g.at[p]).wait()

    def start_write(i, p):
        base = ebase + i * CHUNK_G
        pltpu.async_copy(abuf.at[p], out_hbm.at[pl.ds(base, CHUNK_G)],
                         sem_w.at[p])

    def drain_write(p):
        pltpu.make_async_copy(abuf.at[p], out_hbm.at[pl.ds(0, CHUNK_G)],
                              sem_w.at[p]).wait()

    def compute(p):
        def row_body(r, _):
            for cc in range(CH // 16):
                a = abuf[p, r, pl.ds(cc * 16, 16)]
                b = bbuf[p, r, pl.ds(cc * 16, 16)]
                abuf[p, r, pl.ds(cc * 16, 16)] = 1.0 / (1.0 + a * b)
            return 0

        lax.fori_loop(0, CHUNK_G, row_body, 0)

    def body(i, p, first, start_next, prefetch):
        q = 1 - p
        if start_next:
            if not first:
                drain_write(q)
            drain_idx(q)
            start_gathers(q)
        drain_gathers(p)
        if prefetch:
            start_idx(i + 2, p)
        compute(p)
        start_write(i, p)

    start_idx(0, 0)
    drain_idx(0)
    start_gathers(0)
    start_idx(1, 1)
    body(jnp.int32(0), 0, True, True, True)

    def pair_body(k, _):
        i = 2 * k + 1
        body(i, 1, False, True, True)
        body(i + 1, 0, False, True, True)
        return 0

    lax.fori_loop(0, (NCHUNK_G - 3) // 2, pair_body, 0)
    body(jnp.int32(NCHUNK_G - 2), 1, False, True, False)
    body(jnp.int32(NCHUNK_G - 1), 0, False, False, False)
    drain_write(1)
    drain_write(0)


def kernel(x_1, edge_index, W1, W2):
    src = edge_index[0]
    dst = edge_index[1]
    pagg, pdeg = _sc_scatter(x_1, src, dst)
    g = _tc_middle(pagg, pdeg, W1, W2)
    return _sc_gather(g, src, dst)


# speedup vs baseline: 7.7469x; 1.0953x over previous
import functools

import jax
import jax.numpy as jnp
from jax import lax
from jax.experimental import pallas as pl
from jax.experimental.pallas import tpu as pltpu
from jax.experimental.pallas import tpu_sc as plsc

N_NODES = 10000
N_EDGES = 320000
CH = 128
NC = 2
NS = 16
NW = NC * NS
EW = N_EDGES // NW
CHUNK = 240
NFULL = 41
TAIL = EW - NFULL * CHUNK
CHUNK_G = 80
NCHUNK_G = EW // CHUNK_G
N_PAD = 10240
NR = N_PAD // NS
DR = N_PAD // CH

_mesh = plsc.VectorSubcoreMesh(core_axis_name="c", subcore_axis_name="s")


def _zero_vmem_2d(ref, rows, cols):
    zv = jnp.zeros((16,), jnp.float32)
    cpr = cols // 16

    def body(k, _):
        r = k // cpr
        c = k % cpr
        ref[r, pl.ds(c * 16, 16)] = zv
        return 0

    lax.fori_loop(0, rows * cpr, body, 0)


@functools.partial(
    pl.kernel,
    out_type=(
        jax.ShapeDtypeStruct((NC * N_PAD, CH), jnp.float32),
        jax.ShapeDtypeStruct((NW * DR, CH), jnp.float32),
    ),
    mesh=_mesh,
    scratch_types=[
        pltpu.VMEM_SHARED((N_PAD, CH), jnp.float32),
        pltpu.VMEM((CHUNK, CH), jnp.float32),
        pltpu.VMEM((DR, CH), jnp.float32),
        pltpu.VMEM((CHUNK,), jnp.int32),
        pltpu.VMEM((CHUNK,), jnp.int32),
        pltpu.SemaphoreType.DMA((3,)),
        pltpu.SemaphoreType.DMA((2,)),
    ],
    compiler_params=pltpu.CompilerParams(needs_layout_passes=False),
)
def _sc_scatter(x_hbm, src_hbm, dst_hbm, pagg_hbm, pdeg_hbm,
                agg_s, xbuf, degtab, sidx, didx, sem_l, sem_a):
    c = lax.axis_index("c")
    s = lax.axis_index("s")
    wid = c * NS + s
    nbase = s * NR

    _zero_vmem_2d(degtab, DR, CH)
    _zero_vmem_2d(xbuf, CHUNK, CH)
    pltpu.sync_copy(xbuf, agg_s.at[pl.ds(nbase, CHUNK)])
    pltpu.sync_copy(xbuf, agg_s.at[pl.ds(nbase + CHUNK, CHUNK)])
    pltpu.sync_copy(xbuf.at[pl.ds(0, NR - 2 * CHUNK)],
                    agg_s.at[pl.ds(nbase + 2 * CHUNK, NR - 2 * CHUNK)])
    plsc.subcore_barrier()

    onev = jnp.ones((16,), jnp.float32)

    def count16(idxref, k):
        idx = idxref[pl.ds(k * 16, 16)]
        r16 = lax.shift_right_logical(idx, 7)
        c16 = jnp.bitwise_and(idx, 127)
        plsc.addupdate_scatter(degtab, [r16, c16], onev)

    def do_chunk(base, n):
        sx = sidx.at[pl.ds(0, n)]
        dx = didx.at[pl.ds(0, n)]
        xb = xbuf.at[pl.ds(0, n)]
        pltpu.async_copy(src_hbm.at[pl.ds(base, n)], sx, sem_l.at[0])
        pltpu.async_copy(dst_hbm.at[pl.ds(base, n)], dx, sem_l.at[1])
        pltpu.async_copy(x_hbm.at[pl.ds(base, n)], xb, sem_l.at[2])
        pltpu.make_async_copy(x_hbm.at[pl.ds(base, n)], xb, sem_l.at[2]).wait()
        pltpu.make_async_copy(src_hbm.at[pl.ds(base, n)], sx,
                              sem_l.at[0]).wait()
        pltpu.async_copy(xb, agg_s.at[sx], sem_a.at[0], add=True)
        pltpu.make_async_copy(dst_hbm.at[pl.ds(base, n)], dx,
                              sem_l.at[1]).wait()
        pltpu.async_copy(xb, agg_s.at[dx], sem_a.at[1], add=True)

        def count_body(k, _):
            count16(sidx, k)
            count16(didx, k)
            return 0

        lax.fori_loop(0, n // 16, count_body, 0)
        pltpu.make_async_copy(xb, agg_s.at[sx], sem_a.at[0]).wait()
        pltpu.make_async_copy(xb, agg_s.at[dx], sem_a.at[1]).wait()

    def chunk_body(i, _):
        do_chunk(wid * EW + i * CHUNK, CHUNK)
        return 0

    lax.fori_loop(0, NFULL, chunk_body, 0)
    do_chunk(wid * EW + NFULL * CHUNK, TAIL)
    plsc.subcore_barrier()

    pltpu.sync_copy(agg_s.at[pl.ds(nbase, NR)],
                    pagg_hbm.at[pl.ds(c * N_PAD + nbase, NR)])
    pltpu.sync_copy(degtab, pdeg_hbm.at[pl.ds(wid * DR, DR)])


def _tc_middle_body(pagg_ref, pdeg_ref, w1_ref, w2_ref, g_ref):
    agg = pagg_ref[0:N_NODES, :] + pagg_ref[N_PAD:N_PAD + N_NODES, :]
    degp = pdeg_ref[0:DR, :]
    for w in range(1, NW):
        degp = degp + pdeg_ref[w * DR:(w + 1) * DR, :]
    deg = jnp.reshape(degp, (N_PAD,))[0:N_NODES][:, None]
    deg = jnp.where(deg > 0.0, deg, 1.0)
    z = jax.lax.dot(agg, w1_ref[...], precision=jax.lax.Precision.HIGHEST,
                    preferred_element_type=jnp.float32) / deg
    x0 = 1.0 / (1.0 + jnp.exp(-z))
    m2 = jax.lax.dot(x0, w2_ref[...], precision=jax.lax.Precision.HIGHEST,
                     preferred_element_type=jnp.float32)
    g_ref[0:N_NODES, :] = jnp.exp(m2 * -0.5)
    g_ref[N_NODES:N_PAD, :] = jnp.zeros((N_PAD - N_NODES, CH), jnp.float32)


def _tc_middle(pagg, pdeg, W1, W2):
    return pl.pallas_call(
        _tc_middle_body,
        out_shape=jax.ShapeDtypeStruct((N_PAD, CH), jnp.float32),
    )(pagg, pdeg, W1, W2)


@functools.partial(
    pl.kernel,
    out_type=jax.ShapeDtypeStruct((N_EDGES, CH), jnp.float32),
    mesh=_mesh,
    scratch_types=[
        pltpu.VMEM_SHARED((N_PAD, CH), jnp.float32),
        pltpu.VMEM((2, CHUNK_G), jnp.int32),
        pltpu.VMEM((2, CHUNK_G), jnp.int32),
        pltpu.VMEM((2, CHUNK_G, CH), jnp.float32),
        pltpu.VMEM((2, CHUNK_G, CH), jnp.float32),
        pltpu.SemaphoreType.DMA((2,)),
        pltpu.SemaphoreType.DMA((2,)),
        pltpu.SemaphoreType.DMA((2,)),
    ],
    compiler_params=pltpu.CompilerParams(needs_layout_passes=False),
)
def _sc_gather(g_hbm, src_hbm, dst_hbm, out_hbm,
               g_s, sidx, didx, abuf, bbuf, sem_g, sem_i, sem_w):
    c = lax.axis_index("c")
    s = lax.axis_index("s")
    wid = c * NS + s
    nbase = s * NR
    ebase = wid * EW

    pltpu.sync_copy(g_hbm.at[pl.ds(nbase, NR)], g_s.at[pl.ds(nbase, NR)])
    plsc.subcore_barrier()

    def start_idx(i, p):
        base = ebase + i * CHUNK_G
        pltpu.async_copy(src_hbm.at[pl.ds(base, CHUNK_G)], sidx.at[p],
                         sem_i.at[p])
        pltpu.async_copy(dst_hbm.at[pl.ds(base, CHUNK_G)], didx.at[p],
                         sem_i.at[p])

    def drain_idx(p):
        pltpu.make_async_copy(src_hbm.at[pl.ds(0, CHUNK_G)], sidx.at[p],
                              sem_i.at[p]).wait()
        pltpu.make_async_copy(dst_hbm.at[pl.ds(0, CHUNK_G)], didx.at[p],
                              sem_i.at[p]).wait()

    def start_gathers(p):
        pltpu.async_copy(g_s.at[sidx.at[p]], abuf.at[p], sem_g.at[p])
        pltpu.async_copy(g_s.at[didx.at[p]], bbuf.at[p], sem_g.at[p])

    def drain_gathers(p):
        pltpu.make_async_copy(g_s.at[sidx.at[p]], abuf.at[p],
                              sem_g.at[p]).wait()
        pltpu.make_async_copy(g_s.at[didx.at[p]], bbuf.at[p],
                              sem_g.at[p]).wait()

    def start_write(i, p):
        base = ebase + i * CHUNK_G
        pltpu.async_copy(abuf.at[p], out_hbm.at[pl.ds(base, CHUNK_G)],
                         sem_w.at[p])

    def drain_write(p):
        pltpu.make_async_copy(abuf.at[p], out_hbm.at[pl.ds(0, CHUNK_G)],
                              sem_w.at[p]).wait()

    def compute(p):
        def row_body(r, _):
            for cc in range(CH // 16):
                a = abuf[p, r, pl.ds(cc * 16, 16)]
                b = bbuf[p, r, pl.ds(cc * 16, 16)]
                abuf[p, r, pl.ds(cc * 16, 16)] = 1.0 / (1.0 + a * b)
            return 0

        lax.fori_loop(0, CHUNK_G, row_body, 0)

    def body(i, p, first, start_next, prefetch):
        q = 1 - p
        if start_next:
            if not first:
                drain_write(q)
            drain_idx(q)
            start_gathers(q)
        drain_gathers(p)
        if prefetch:
            start_idx(i + 2, p)
        compute(p)
        start_write(i, p)

    start_idx(0, 0)
    drain_idx(0)
    start_gathers(0)
    start_idx(1, 1)
    body(jnp.int32(0), 0, True, True, True)

    def pair_body(k, _):
        i = 2 * k + 1
        body(i, 1, False, True, True)
        body(i + 1, 0, False, True, True)
        return 0

    lax.fori_loop(0, (NCHUNK_G - 3) // 2, pair_body, 0)
    body(jnp.int32(NCHUNK_G - 2), 1, False, True, False)
    body(jnp.int32(NCHUNK_G - 1), 0, False, False, False)
    drain_write(1)
    drain_write(0)


def kernel(x_1, edge_index, W1, W2):
    src = edge_index[0]
    dst = edge_index[1]
    pagg, pdeg = _sc_scatter(x_1, src, dst)
    g = _tc_middle(pagg, pdeg, W1, W2)
    return _sc_gather(g, src, dst)
